# baseline (speedup 1.0000x reference)
import functools

import jax
import jax.numpy as jnp
from jax.experimental import pallas as pl
from jax.experimental.pallas import tpu as pltpu

_B = 64
_T = 16384
_ROWS = 8
_PAD = 128
_N = _T + 2 * _PAD
_NEG = float("-inf")


def _sr(x, k):
    return pltpu.roll(x, k, axis=1)


def _sl(x, k):
    return pltpu.roll(x, _N - k, axis=1)


def _peak_mask(q):
    f2 = jnp.maximum(q, _sr(q, 1))
    f4 = jnp.maximum(f2, _sr(f2, 2))
    f8 = jnp.maximum(f4, _sr(f4, 4))
    f12 = jnp.maximum(f8, _sr(f4, 8))
    return (q > _sr(f12, 1)) & (q >= _sl(f12, 12))


def _onsets_kernel(p_ref, l_ref, out_ref):
    i = pl.program_id(0)

    @pl.when(i == 0)
    def _init():
        out_ref[0, 0] = jnp.int32(0)

    pad = jnp.full((_ROWS, _PAD), _NEG, jnp.float32)
    p = jnp.concatenate([pad, p_ref[...], pad], axis=1)
    l = jnp.concatenate([pad, l_ref[...], pad], axis=1)
    sp = _peak_mask(p)
    sl = _peak_mask(l)
    lane = jax.lax.broadcasted_iota(jnp.int16, sp.shape, 1)
    d = jnp.where(
        lane >= _PAD + 1,
        sp.astype(jnp.int16) - sl.astype(jnp.int16),
        jnp.int16(0),
    )
    d = d[:, _PAD:_PAD + _T]
    lane = lane[:, _PAD:_PAD + _T]
    s = d
    k = 1
    while k < _T:
        r = pltpu.roll(s, k, axis=1)
        s = s + jnp.where(lane < k + _PAD, jnp.int16(0), r)
        k *= 2
    total = s[:, -1:]
    block_sum = jnp.sum(jnp.abs(total - s).astype(jnp.int32))
    out_ref[0, 0] += block_sum


@jax.jit
def kernel(preds, labels):
    grid = _B // _ROWS
    acc = pl.pallas_call(
        _onsets_kernel,
        grid=(grid,),
        in_specs=[
            pl.BlockSpec((_ROWS, _T), lambda i: (i, 0)),
            pl.BlockSpec((_ROWS, _T), lambda i: (i, 0)),
        ],
        out_specs=pl.BlockSpec(memory_space=pltpu.SMEM),
        out_shape=jax.ShapeDtypeStruct((1, 1), jnp.int32),
    )(preds, labels)
    return acc[0, 0].astype(jnp.float32) / jnp.float32(_B * _T)


# speedup vs baseline: 1.0915x; 1.0915x over previous
import functools

import jax
import jax.numpy as jnp
from jax.experimental import pallas as pl
from jax.experimental.pallas import tpu as pltpu

_B = 64
_T = 16384
_ROWS = 8
_PAD = 128
_N = _T + 2 * _PAD
_NEG = float("-inf")


def _sr(x, k):
    return pltpu.roll(x, k, axis=1)


def _sl(x, k):
    return pltpu.roll(x, _N - k, axis=1)


def _peak_mask(q):
    f2 = jnp.maximum(q, _sr(q, 1))
    f4 = jnp.maximum(f2, _sr(f2, 2))
    f8 = jnp.maximum(f4, _sr(f4, 4))
    f12 = jnp.maximum(f8, _sr(f4, 8))
    return (q > _sr(f12, 1)) & (q >= _sl(f12, 12))


def _onsets_kernel(p_ref, l_ref, out_ref):
    i = pl.program_id(0)

    @pl.when(i == 0)
    def _init():
        out_ref[0, 0] = jnp.int32(0)

    pad = jnp.full((_ROWS, _PAD), _NEG, jnp.float32)
    p = jnp.concatenate([pad, p_ref[...], pad], axis=1)
    l = jnp.concatenate([pad, l_ref[...], pad], axis=1)
    sp = _peak_mask(p)
    sl = _peak_mask(l)
    lane = jax.lax.broadcasted_iota(jnp.int32, sp.shape, 1)
    d = jnp.where(lane >= _PAD + 1, sp.astype(jnp.int32) - sl.astype(jnp.int32), 0)
    d = d[:, _PAD:_PAD + _T]
    lane = lane[:, _PAD:_PAD + _T]
    s = d
    k = 1
    while k < _T:
        r = pltpu.roll(s, k, axis=1)
        s = s + jnp.where(lane < k + _PAD, 0, r)
        k *= 2
    total = s[:, -1:]
    block_sum = jnp.sum(jnp.abs(total - s))
    out_ref[0, 0] += block_sum


@jax.jit
def kernel(preds, labels):
    grid = _B // _ROWS
    acc = pl.pallas_call(
        _onsets_kernel,
        grid=(grid,),
        in_specs=[
            pl.BlockSpec((_ROWS, _T), lambda i: (i, 0)),
            pl.BlockSpec((_ROWS, _T), lambda i: (i, 0)),
        ],
        out_specs=pl.BlockSpec(memory_space=pltpu.SMEM),
        out_shape=jax.ShapeDtypeStruct((1, 1), jnp.int32),
    )(preds, labels)
    return acc[0, 0].astype(jnp.float32) / jnp.float32(_B * _T)
